# baseline (speedup 1.0000x reference)
import functools

import jax
import jax.numpy as jnp
from jax import lax
from jax.experimental import pallas as pl
from jax.experimental.pallas import tpu as pltpu
from jax.experimental.pallas import tpu_sc as plsc

D = 128
N_NODES = 10000
NP = 10240
NE = 160000
EP = 163840
NC, NS = 2, 16
NW = NC * NS
EPW = EP // NW
CH = 128
NCHUNK = EPW // CH
RPS = NP // NS


def _dot(a, b):
    return lax.dot_general(a.astype(jnp.bfloat16), b.astype(jnp.bfloat16),
                           (((1,), (0,)), ((), ())),
                           preferred_element_type=jnp.float32)


def _ln(y, g, o):
    m = jnp.mean(y, axis=-1, keepdims=True)
    d = y - m
    v = jnp.mean(d * d, axis=-1, keepdims=True)
    return d * lax.rsqrt(v + 1e-5) * g + o


def _row_spec(blk, d):
    return pl.BlockSpec((blk, d), lambda i: (i, 0))


def _full_spec(shape):
    return pl.BlockSpec(shape, lambda i: (0,) * len(shape))


def _block_args(block):
    mlp = block["mlp"]
    return (mlp[0]["W"], mlp[0]["b"].reshape(1, D),
            mlp[1]["W"], mlp[1]["b"].reshape(1, D),
            block["ln_scale"].reshape(1, D), block["ln_offset"].reshape(1, D))


def _embed_body(x_ref, w1, b1, w2, b2, g, o, out_ref):
    h = jnp.maximum(_dot(x_ref[...], w1[...]) + b1[...], 0.0)
    y = _dot(h, w2[...]) + b2[...]
    out_ref[...] = _ln(y, g[...], o[...])


def _embed(x, blk, w1, b1, w2, b2, g, o):
    n, din = x.shape
    return pl.pallas_call(
        _embed_body,
        grid=(n // blk,),
        in_specs=[_row_spec(blk, din)] + [_full_spec(a.shape)
                                          for a in (w1, b1, w2, b2, g, o)],
        out_specs=_row_spec(blk, D),
        out_shape=jax.ShapeDtypeStruct((n, D), jnp.float32),
    )(x, w1, b1, w2, b2, g, o)


def _edge_body_res(e_ref, s_ref, r_ref, wa, wb, wc, b1, w2, b2, g, o,
                   upd_ref, new_ref):
    h = jnp.maximum(_dot(e_ref[...], wa[...]) + _dot(s_ref[...], wb[...])
                    + _dot(r_ref[...], wc[...]) + b1[...], 0.0)
    y = _ln(_dot(h, w2[...]) + b2[...], g[...], o[...])
    upd_ref[...] = y
    new_ref[...] = e_ref[...] + y


def _edge_body_last(e_ref, s_ref, r_ref, wa, wb, wc, b1, w2, b2, g, o,
                    upd_ref):
    h = jnp.maximum(_dot(e_ref[...], wa[...]) + _dot(s_ref[...], wb[...])
                    + _dot(r_ref[...], wc[...]) + b1[...], 0.0)
    upd_ref[...] = _ln(_dot(h, w2[...]) + b2[...], g[...], o[...])


def _edge_update(edges, sent, recv, block, want_residual, blk=2048):
    n = edges.shape[0]
    w1, b1, w2, b2, g, o = _block_args(block)
    wa, wb, wc = w1[:D], w1[D:2 * D], w1[2 * D:]
    weights = (wa, wb, wc, b1, w2, b2, g, o)
    out_shape = jax.ShapeDtypeStruct((n, D), jnp.float32)
    if want_residual:
        body = _edge_body_res
        outs = (out_shape, out_shape)
        ospec = (_row_spec(blk, D), _row_spec(blk, D))
    else:
        body = _edge_body_last
        outs = out_shape
        ospec = _row_spec(blk, D)
    return pl.pallas_call(
        body,
        grid=(n // blk,),
        in_specs=[_row_spec(blk, D)] * 3 + [_full_spec(a.shape)
                                            for a in weights],
        out_specs=ospec,
        out_shape=outs,
    )(edges, sent, recv, *weights)


def _node_body(n_ref, a0_ref, a1_ref, a2_ref, a3_ref,
               wn, wa, b1, w2, b2, g, o, out_ref):
    agg = (a0_ref[...] + a1_ref[...]) + (a2_ref[...] + a3_ref[...])
    h = jnp.maximum(_dot(n_ref[...], wn[...]) + _dot(agg, wa[...])
                    + b1[...], 0.0)
    y = _ln(_dot(h, w2[...]) + b2[...], g[...], o[...])
    out_ref[...] = n_ref[...] + y


def _node_update(nodes, aggs, block, blk=2048):
    w1, b1, w2, b2, g, o = _block_args(block)
    wn, wa = w1[:D], w1[D:]
    weights = (wn, wa, b1, w2, b2, g, o)
    return pl.pallas_call(
        _node_body,
        grid=(NP // blk,),
        in_specs=[_row_spec(blk, D)] * 5 + [_full_spec(a.shape)
                                            for a in weights],
        out_specs=_row_spec(blk, D),
        out_shape=jax.ShapeDtypeStruct((NP, D), jnp.float32),
    )(nodes, *aggs, *weights)


def _decode_body(x_ref, w1, b1, w2, b2, out_ref):
    h = jnp.maximum(_dot(x_ref[...], w1[...]) + b1[...], 0.0)
    out_ref[...] = _dot(h, w2[...]) + b2[...]


def _decode(nodes, mlp, blk=2048):
    weights = (mlp[0]["W"], mlp[0]["b"].reshape(1, D),
               mlp[1]["W"], mlp[1]["b"].reshape(1, D))
    return pl.pallas_call(
        _decode_body,
        grid=(NP // blk,),
        in_specs=[_row_spec(blk, D)] + [_full_spec(a.shape) for a in weights],
        out_specs=_row_spec(blk, D),
        out_shape=jax.ShapeDtypeStruct((NP, D), jnp.float32),
    )(nodes, *weights)


_MESH = plsc.VectorSubcoreMesh(core_axis_name="c", subcore_axis_name="s")

NBUF = 4
GCH = 80


def _sc_gather(nodes, sidx, ridx):
    n = sidx.shape[0]
    epw = n // NW
    gnch = epw // GCH
    assert epw % GCH == 0 and gnch % NBUF == 0

    @functools.partial(
        pl.kernel,
        out_type=[jax.ShapeDtypeStruct((n, D), jnp.float32),
                  jax.ShapeDtypeStruct((n, D), jnp.float32)],
        mesh=_MESH,
        scratch_types=[pltpu.VMEM((epw,), jnp.int32),
                       pltpu.VMEM((epw,), jnp.int32),
                       pltpu.VMEM((NBUF, GCH, D), jnp.float32),
                       pltpu.VMEM((NBUF, GCH, D), jnp.float32)]
                      + [pltpu.SemaphoreType.DMA] * (2 * NBUF),
    )
    def k(nodes_hbm, sidx_hbm, ridx_hbm, sent_hbm, recv_hbm,
          idxs_v, idxr_v, rs_v, rr_v, *sems):
        sg = sems[:NBUF]
        so = sems[NBUF:]
        s_ax = lax.axis_index("s")
        w = lax.axis_index("c") * NS + s_ax
        base = w * epw
        pltpu.sync_copy(sidx_hbm.at[pl.ds(base, epw)], idxs_v)
        pltpu.sync_copy(ridx_hbm.at[pl.ds(base, epw)], idxr_v)

        @pl.loop(0, RPS // GCH)
        def _(i):
            pltpu.async_copy(
                nodes_hbm.at[pl.ds(s_ax * RPS + i * GCH, GCH)],
                rs_v.at[0], sg[0])

        @pl.loop(0, RPS // GCH)
        def _(i):
            pltpu.make_async_copy(nodes_hbm.at[pl.ds(0, GCH)],
                                  rs_v.at[0], sg[0]).wait()

        plsc.subcore_barrier()

        def g_start(c, b):
            off = c * GCH
            pltpu.async_copy(nodes_hbm.at[idxs_v.at[pl.ds(off, GCH)]],
                             rs_v.at[b], sg[b])
            pltpu.async_copy(nodes_hbm.at[idxr_v.at[pl.ds(off, GCH)]],
                             rr_v.at[b], sg[b])

        def g_wait(b):
            pltpu.make_async_copy(nodes_hbm.at[idxs_v.at[pl.ds(0, GCH)]],
                                  rs_v.at[b], sg[b]).wait()
            pltpu.make_async_copy(nodes_hbm.at[idxr_v.at[pl.ds(0, GCH)]],
                                  rr_v.at[b], sg[b]).wait()

        def o_start(c, b):
            off = base + c * GCH
            pltpu.async_copy(rs_v.at[b], sent_hbm.at[pl.ds(off, GCH)], so[b])
            pltpu.async_copy(rr_v.at[b], recv_hbm.at[pl.ds(off, GCH)], so[b])

        def o_wait(b):
            pltpu.make_async_copy(rs_v.at[b], sent_hbm.at[pl.ds(base, GCH)],
                                  so[b]).wait()
            pltpu.make_async_copy(rr_v.at[b], recv_hbm.at[pl.ds(base, GCH)],
                                  so[b]).wait()

        g_start(0, 0)
        g_start(1, 1)

        @pl.loop(0, gnch, step=NBUF)
        def _(c0):
            for j in range(NBUF):
                b = j % NBUF
                bn = (j + 2) % NBUF
                cc = c0 + j
                g_wait(b)
                o_start(cc, b)
                @pl.when(cc >= 2)
                def _():
                    o_wait(bn)

                @pl.when(cc + 2 < gnch)
                def _():
                    g_start(cc + 2, bn)

        o_wait((gnch - 2) % NBUF)
        o_wait((gnch - 1) % NBUF)

    return k(nodes, sidx, ridx)


def _sc_scatter(e_upd, ridx):
    n = ridx.shape[0]
    epw = n // NW
    nchunk = epw // CH
    assert epw % CH == 0

    @functools.partial(
        pl.kernel,
        out_type=jax.ShapeDtypeStruct((NC, NP, D), jnp.float32),
        mesh=_MESH,
        scratch_types=[pltpu.VMEM_SHARED((NP, D), jnp.float32),
                       pltpu.VMEM((CH, D), jnp.float32),
                       pltpu.VMEM((CH,), jnp.int32),
                       pltpu.VMEM((CH, D), jnp.float32)],
    )
    def k(e_hbm, ridx_hbm, out_hbm, shared, ebuf, idx_v, zbuf):
        c = lax.axis_index("c")
        s = lax.axis_index("s")

        @pl.loop(0, CH)
        def _(i):
            @pl.loop(0, D // 16)
            def _(j):
                zbuf[i, pl.ds(j * 16, 16)] = jnp.zeros((16,), jnp.float32)

        @pl.loop(0, RPS // CH)
        def _(kk):
            pltpu.sync_copy(zbuf, shared.at[pl.ds(s * RPS + kk * CH, CH)])

        plsc.subcore_barrier()
        base = (c * NS + s) * epw

        @pl.loop(0, nchunk)
        def _(i):
            off = base + i * CH
            pltpu.sync_copy(ridx_hbm.at[pl.ds(off, CH)], idx_v)
            pltpu.sync_copy(e_hbm.at[pl.ds(off, CH)], ebuf)
            pltpu.sync_copy(ebuf, shared.at[idx_v], add=True)

        plsc.subcore_barrier()
        pltpu.sync_copy(shared.at[pl.ds(s * RPS, RPS)],
                        out_hbm.at[c, pl.ds(s * RPS, RPS)])

    return k(e_upd, ridx)


EH = EP // 2


def kernel(x, edge_attr, edge_index, params):
    f32 = jnp.float32
    senders = edge_index[0].astype(jnp.int32)
    receivers = edge_index[1].astype(jnp.int32)

    xp = jnp.zeros((NP, x.shape[1]), f32).at[:N_NODES].set(x)
    eap = jnp.zeros((EP, edge_attr.shape[1]), f32).at[:NE].set(edge_attr)
    sidx = jnp.zeros((EP,), jnp.int32).at[:NE].set(senders)
    ridx_g = jnp.zeros((EP,), jnp.int32).at[:NE].set(receivers)
    ridx_s = jnp.full((EP,), N_NODES + 8, jnp.int32).at[:NE].set(receivers)

    halves = [slice(0, EH), slice(EH, EP)]
    sidx_h = [sidx[h] for h in halves]
    ridx_gh = [ridx_g[h] for h in halves]
    ridx_sh = [ridx_s[h] for h in halves]

    nodes = _embed(xp, 2048, *_block_args(params["embed_node"]))
    edges = [_embed(eap[h], 2048, *_block_args(params["embed_edge"]))
             for h in halves]

    nsteps = len(params["steps"])
    for si, sp in enumerate(params["steps"]):
        last = si + 1 == nsteps
        gathered = [_sc_gather(nodes, sidx_h[i], ridx_gh[i])
                    for i in range(2)]
        aggs = []
        new_edges = []
        for i in range(2):
            sent, recv = gathered[i]
            if last:
                e_upd = _edge_update(edges[i], sent, recv, sp["edge"], False)
            else:
                e_upd, e_new = _edge_update(edges[i], sent, recv,
                                            sp["edge"], True)
                new_edges.append(e_new)
            agg = _sc_scatter(e_upd, ridx_sh[i])
            aggs.extend([agg[0], agg[1]])
        edges = new_edges
        nodes = _node_update(nodes, aggs, sp["node"])

    out = _decode(nodes, params["decode_node"]["mlp"])
    return out[:N_NODES]


# speedup vs baseline: 1.0519x; 1.0519x over previous
import functools

import jax
import jax.numpy as jnp
from jax import lax
from jax.experimental import pallas as pl
from jax.experimental.pallas import tpu as pltpu
from jax.experimental.pallas import tpu_sc as plsc

D = 128
N_NODES = 10000
NP = 10240
NE = 160000
EP = 163840
NC, NS = 2, 16
NW = NC * NS
EPW = EP // NW
CH = 128
NCHUNK = EPW // CH
RPS = NP // NS


def _dot(a, b):
    return lax.dot_general(a.astype(jnp.bfloat16), b.astype(jnp.bfloat16),
                           (((1,), (0,)), ((), ())),
                           preferred_element_type=jnp.float32)


def _ln(y, g, o):
    m = jnp.mean(y, axis=-1, keepdims=True)
    d = y - m
    v = jnp.mean(d * d, axis=-1, keepdims=True)
    return d * lax.rsqrt(v + 1e-5) * g + o


def _row_spec(blk, d):
    return pl.BlockSpec((blk, d), lambda i: (i, 0))


def _full_spec(shape):
    return pl.BlockSpec(shape, lambda i: (0,) * len(shape))


def _block_args(block):
    mlp = block["mlp"]
    return (mlp[0]["W"], mlp[0]["b"].reshape(1, D),
            mlp[1]["W"], mlp[1]["b"].reshape(1, D),
            block["ln_scale"].reshape(1, D), block["ln_offset"].reshape(1, D))


def _embed_body(x_ref, w1, b1, w2, b2, g, o, out_ref):
    h = jnp.maximum(_dot(x_ref[...], w1[...]) + b1[...], 0.0)
    y = _dot(h, w2[...]) + b2[...]
    out_ref[...] = _ln(y, g[...], o[...])


def _embed(x, blk, w1, b1, w2, b2, g, o):
    n, din = x.shape
    return pl.pallas_call(
        _embed_body,
        grid=(n // blk,),
        in_specs=[_row_spec(blk, din)] + [_full_spec(a.shape)
                                          for a in (w1, b1, w2, b2, g, o)],
        out_specs=_row_spec(blk, D),
        out_shape=jax.ShapeDtypeStruct((n, D), jnp.float32),
    )(x, w1, b1, w2, b2, g, o)


def _edge_body_res(e_ref, s_ref, r_ref, wa, wb, wc, b1, w2, b2, g, o,
                   upd_ref, new_ref):
    h = jnp.maximum(_dot(e_ref[...], wa[...]) + _dot(s_ref[...], wb[...])
                    + _dot(r_ref[...], wc[...]) + b1[...], 0.0)
    y = _ln(_dot(h, w2[...]) + b2[...], g[...], o[...])
    upd_ref[...] = y
    new_ref[...] = e_ref[...] + y


def _edge_body_last(e_ref, s_ref, r_ref, wa, wb, wc, b1, w2, b2, g, o,
                    upd_ref):
    h = jnp.maximum(_dot(e_ref[...], wa[...]) + _dot(s_ref[...], wb[...])
                    + _dot(r_ref[...], wc[...]) + b1[...], 0.0)
    upd_ref[...] = _ln(_dot(h, w2[...]) + b2[...], g[...], o[...])


def _edge_update(edges, sent, recv, block, want_residual, blk=2048):
    n = edges.shape[0]
    w1, b1, w2, b2, g, o = _block_args(block)
    wa, wb, wc = w1[:D], w1[D:2 * D], w1[2 * D:]
    weights = (wa, wb, wc, b1, w2, b2, g, o)
    out_shape = jax.ShapeDtypeStruct((n, D), jnp.float32)
    if want_residual:
        body = _edge_body_res
        outs = (out_shape, out_shape)
        ospec = (_row_spec(blk, D), _row_spec(blk, D))
    else:
        body = _edge_body_last
        outs = out_shape
        ospec = _row_spec(blk, D)
    return pl.pallas_call(
        body,
        grid=(n // blk,),
        in_specs=[_row_spec(blk, D)] * 3 + [_full_spec(a.shape)
                                            for a in weights],
        out_specs=ospec,
        out_shape=outs,
    )(edges, sent, recv, *weights)


def _make_node_body(n_agg):
    def body(*refs):
        n_ref = refs[0]
        aggs = refs[1:1 + n_agg]
        wn, wa, b1, w2, b2, g, o = refs[1 + n_agg:1 + n_agg + 7]
        out_ref = refs[-1]
        agg = aggs[0][...]
        for a in aggs[1:]:
            agg = agg + a[...]
        h = jnp.maximum(_dot(n_ref[...], wn[...]) + _dot(agg, wa[...])
                        + b1[...], 0.0)
        y = _ln(_dot(h, w2[...]) + b2[...], g[...], o[...])
        out_ref[...] = n_ref[...] + y
    return body


def _node_update(nodes, aggs, block, blk=2048):
    w1, b1, w2, b2, g, o = _block_args(block)
    wn, wa = w1[:D], w1[D:]
    weights = (wn, wa, b1, w2, b2, g, o)
    return pl.pallas_call(
        _make_node_body(len(aggs)),
        grid=(NP // blk,),
        in_specs=[_row_spec(blk, D)] * (1 + len(aggs))
                 + [_full_spec(a.shape) for a in weights],
        out_specs=_row_spec(blk, D),
        out_shape=jax.ShapeDtypeStruct((NP, D), jnp.float32),
    )(nodes, *aggs, *weights)


def _decode_body(x_ref, w1, b1, w2, b2, out_ref):
    h = jnp.maximum(_dot(x_ref[...], w1[...]) + b1[...], 0.0)
    out_ref[...] = _dot(h, w2[...]) + b2[...]


def _decode(nodes, mlp, blk=2048):
    weights = (mlp[0]["W"], mlp[0]["b"].reshape(1, D),
               mlp[1]["W"], mlp[1]["b"].reshape(1, D))
    return pl.pallas_call(
        _decode_body,
        grid=(NP // blk,),
        in_specs=[_row_spec(blk, D)] + [_full_spec(a.shape) for a in weights],
        out_specs=_row_spec(blk, D),
        out_shape=jax.ShapeDtypeStruct((NP, D), jnp.float32),
    )(nodes, *weights)


_MESH = plsc.VectorSubcoreMesh(core_axis_name="c", subcore_axis_name="s")

NBUF = 4
GCH = 80


def _sc_gather(nodes, sidx, ridx):
    n = sidx.shape[0]
    epw = n // NW
    gnch = epw // GCH
    assert epw % GCH == 0 and gnch % NBUF == 0

    @functools.partial(
        pl.kernel,
        out_type=[jax.ShapeDtypeStruct((n, D), jnp.float32),
                  jax.ShapeDtypeStruct((n, D), jnp.float32)],
        mesh=_MESH,
        scratch_types=[pltpu.VMEM((epw,), jnp.int32),
                       pltpu.VMEM((epw,), jnp.int32),
                       pltpu.VMEM((NBUF, GCH, D), jnp.float32),
                       pltpu.VMEM((NBUF, GCH, D), jnp.float32)]
                      + [pltpu.SemaphoreType.DMA] * (2 * NBUF),
    )
    def k(nodes_hbm, sidx_hbm, ridx_hbm, sent_hbm, recv_hbm,
          idxs_v, idxr_v, rs_v, rr_v, *sems):
        sg = sems[:NBUF]
        so = sems[NBUF:]
        s_ax = lax.axis_index("s")
        w = lax.axis_index("c") * NS + s_ax
        base = w * epw
        pltpu.sync_copy(sidx_hbm.at[pl.ds(base, epw)], idxs_v)
        pltpu.sync_copy(ridx_hbm.at[pl.ds(base, epw)], idxr_v)

        def g_start(c, b):
            off = c * GCH
            pltpu.async_copy(nodes_hbm.at[idxs_v.at[pl.ds(off, GCH)]],
                             rs_v.at[b], sg[b])
            pltpu.async_copy(nodes_hbm.at[idxr_v.at[pl.ds(off, GCH)]],
                             rr_v.at[b], sg[b])

        def g_wait(b):
            pltpu.make_async_copy(nodes_hbm.at[idxs_v.at[pl.ds(0, GCH)]],
                                  rs_v.at[b], sg[b]).wait()
            pltpu.make_async_copy(nodes_hbm.at[idxr_v.at[pl.ds(0, GCH)]],
                                  rr_v.at[b], sg[b]).wait()

        def o_start(c, b):
            off = base + c * GCH
            pltpu.async_copy(rs_v.at[b], sent_hbm.at[pl.ds(off, GCH)], so[b])
            pltpu.async_copy(rr_v.at[b], recv_hbm.at[pl.ds(off, GCH)], so[b])

        def o_wait(b):
            pltpu.make_async_copy(rs_v.at[b], sent_hbm.at[pl.ds(base, GCH)],
                                  so[b]).wait()
            pltpu.make_async_copy(rr_v.at[b], recv_hbm.at[pl.ds(base, GCH)],
                                  so[b]).wait()

        g_start(0, 0)
        g_start(1, 1)

        @pl.loop(0, gnch, step=NBUF)
        def _(c0):
            for j in range(NBUF):
                b = j % NBUF
                bn = (j + 2) % NBUF
                cc = c0 + j
                g_wait(b)
                o_start(cc, b)
                @pl.when(cc >= 2)
                def _():
                    o_wait(bn)

                @pl.when(cc + 2 < gnch)
                def _():
                    g_start(cc + 2, bn)

        o_wait((gnch - 2) % NBUF)
        o_wait((gnch - 1) % NBUF)

    return k(nodes, sidx, ridx)


def _sc_scatter(e_upd, ridx):
    n = ridx.shape[0]
    epw = n // NW
    nchunk = epw // CH
    assert epw % CH == 0

    @functools.partial(
        pl.kernel,
        out_type=jax.ShapeDtypeStruct((NC, NP, D), jnp.float32),
        mesh=_MESH,
        scratch_types=[pltpu.VMEM_SHARED((NP, D), jnp.float32),
                       pltpu.VMEM((CH, D), jnp.float32),
                       pltpu.VMEM((CH,), jnp.int32),
                       pltpu.VMEM((CH, D), jnp.float32)],
    )
    def k(e_hbm, ridx_hbm, out_hbm, shared, ebuf, idx_v, zbuf):
        c = lax.axis_index("c")
        s = lax.axis_index("s")

        @pl.loop(0, CH)
        def _(i):
            @pl.loop(0, D // 16)
            def _(j):
                zbuf[i, pl.ds(j * 16, 16)] = jnp.zeros((16,), jnp.float32)

        @pl.loop(0, RPS // CH)
        def _(kk):
            pltpu.sync_copy(zbuf, shared.at[pl.ds(s * RPS + kk * CH, CH)])

        plsc.subcore_barrier()
        base = (c * NS + s) * epw

        @pl.loop(0, nchunk)
        def _(i):
            off = base + i * CH
            pltpu.sync_copy(ridx_hbm.at[pl.ds(off, CH)], idx_v)
            pltpu.sync_copy(e_hbm.at[pl.ds(off, CH)], ebuf)
            pltpu.sync_copy(ebuf, shared.at[idx_v], add=True)

        plsc.subcore_barrier()
        pltpu.sync_copy(shared.at[pl.ds(s * RPS, RPS)],
                        out_hbm.at[c, pl.ds(s * RPS, RPS)])

    return k(e_upd, ridx)


NSPLIT = 4
EH = EP // NSPLIT


def kernel(x, edge_attr, edge_index, params):
    f32 = jnp.float32
    senders = edge_index[0].astype(jnp.int32)
    receivers = edge_index[1].astype(jnp.int32)

    xp = jnp.zeros((NP, x.shape[1]), f32).at[:N_NODES].set(x)
    eap = jnp.zeros((EP, edge_attr.shape[1]), f32).at[:NE].set(edge_attr)
    sidx = jnp.zeros((EP,), jnp.int32).at[:NE].set(senders)
    ridx_g = jnp.zeros((EP,), jnp.int32).at[:NE].set(receivers)
    ridx_s = jnp.full((EP,), N_NODES + 8, jnp.int32).at[:NE].set(receivers)

    splits = [slice(i * EH, (i + 1) * EH) for i in range(NSPLIT)]
    sidx_h = [sidx[h] for h in splits]
    ridx_gh = [ridx_g[h] for h in splits]
    ridx_sh = [ridx_s[h] for h in splits]

    nodes = _embed(xp, 2048, *_block_args(params["embed_node"]))
    edges = [_embed(eap[h], 2048, *_block_args(params["embed_edge"]))
             for h in splits]

    nsteps = len(params["steps"])
    for si, sp in enumerate(params["steps"]):
        last = si + 1 == nsteps
        gathered = [_sc_gather(nodes, sidx_h[i], ridx_gh[i])
                    for i in range(NSPLIT)]
        aggs = []
        new_edges = []
        for i in range(NSPLIT):
            sent, recv = gathered[i]
            if last:
                e_upd = _edge_update(edges[i], sent, recv, sp["edge"], False)
            else:
                e_upd, e_new = _edge_update(edges[i], sent, recv,
                                            sp["edge"], True)
                new_edges.append(e_new)
            agg = _sc_scatter(e_upd, ridx_sh[i])
            aggs.extend([agg[0], agg[1]])
        edges = new_edges
        nodes = _node_update(nodes, aggs, sp["node"])

    out = _decode(nodes, params["decode_node"]["mlp"])
    return out[:N_NODES]
